# baseline (speedup 1.0000x reference)
import functools

import jax
import jax.numpy as jnp
from jax import lax
from jax.experimental import pallas as pl
from jax.experimental.pallas import tpu as pltpu
from jax.experimental.pallas import tpu_sc as plsc

_N = 10000
_E = 320000
_D = 128
_DP = 144
_DG = 16
_NC = 2
_NS = 16
_NW = _NC * _NS
_EP = 327680
_EPW = _EP // _NW
_B = 128
_NCH = _EPW // _B
_NPAIR = _NCH // 2
_NP = 10112
_RPT = _NP // _NS

_mesh = plsc.VectorSubcoreMesh(core_axis_name="c", subcore_axis_name="s")


@functools.partial(
    pl.kernel,
    out_type=(
        jax.ShapeDtypeStruct((_NC, _NP, _D), jnp.float32),
        jax.ShapeDtypeStruct((_NC, _NP, _DG), jnp.float32),
    ),
    mesh=_mesh,
    scratch_types=[
        pltpu.VMEM_SHARED((_NP, _DP), jnp.float32),
        pltpu.VMEM((8, _B), jnp.int32),
        pltpu.VMEM((_B, _DP), jnp.float32),
        pltpu.VMEM((_B, _DP), jnp.float32),
        pltpu.SemaphoreType.DMA,
        pltpu.SemaphoreType.DMA,
        pltpu.SemaphoreType.DMA,
        pltpu.SemaphoreType.DMA,
        pltpu.SemaphoreType.DMA,
    ],
    compiler_params=pltpu.CompilerParams(use_tc_tiling_on_sc=False),
)
def _sc_aggregate(xa, eidx, out_f, out_d, acc, idxb, rows0, rows1,
                  gsem0, gsem1, ssem0, ssem1, isem):
    c = lax.axis_index("c")
    s = lax.axis_index("s")
    wid = s * _NC + c
    off = pl.multiple_of(s * _RPT, 8)

    zero16 = jnp.zeros((16,), jnp.float32)

    def zrows_row(r, carry):
        for k in range(_DP // 16):
            rows0[r, pl.ds(16 * k, 16)] = zero16
        return carry

    lax.fori_loop(0, _B, zrows_row, 0)

    def zero_blk(z, carry):
        zoff = pl.multiple_of(off + z * _B, 8)
        pltpu.sync_copy(rows0, acc.at[pl.ds(zoff, _B)])
        return carry

    lax.fori_loop(0, 4, zero_blk, 0)
    tail = pl.multiple_of(off + 4 * _B, 8)
    pltpu.sync_copy(rows0.at[pl.ds(0, _RPT - 4 * _B)],
                    acc.at[pl.ds(tail, _RPT - 4 * _B)])
    plsc.subcore_barrier()

    pltpu.async_copy(eidx.at[wid, 0], idxb.at[pl.ds(0, 4)], isem).wait()
    pltpu.async_copy(xa.at[idxb.at[0]], rows0, gsem0)
    pltpu.async_copy(xa.at[idxb.at[1]], rows1, gsem1)
    pltpu.async_copy(eidx.at[wid, 1], idxb.at[pl.ds(4, 4)], isem)

    def body(i, carry):
        p = lax.rem(i, 2)
        b0 = 4 * p
        bn = 4 * (1 - p)
        pltpu.make_async_copy(xa.at[idxb.at[b0]], rows0, gsem0).wait()
        pltpu.async_copy(rows0, acc.at[idxb.at[b0 + 2]], ssem0, add=True)
        pltpu.make_async_copy(xa.at[idxb.at[b0 + 1]], rows1, gsem1).wait()
        pltpu.async_copy(rows1, acc.at[idxb.at[b0 + 3]], ssem1, add=True)

        pltpu.make_async_copy(rows0, acc.at[idxb.at[b0 + 2]], ssem0).wait()

        @pl.when(i < _NPAIR - 1)
        def _():
            pltpu.make_async_copy(
                eidx.at[wid, i + 1], idxb.at[pl.ds(bn, 4)], isem).wait()
            pltpu.async_copy(xa.at[idxb.at[bn]], rows0, gsem0)

        pltpu.make_async_copy(rows1, acc.at[idxb.at[b0 + 3]], ssem1).wait()

        @pl.when(i < _NPAIR - 1)
        def _():
            pltpu.async_copy(xa.at[idxb.at[bn + 1]], rows1, gsem1)

        @pl.when(i < _NPAIR - 2)
        def _():
            pltpu.async_copy(eidx.at[wid, i + 2], idxb.at[pl.ds(b0, 4)], isem)

        return carry

    lax.fori_loop(0, _NPAIR, body, 0)
    plsc.subcore_barrier()
    pltpu.sync_copy(acc.at[pl.ds(off, _RPT), pl.ds(0, _D)],
                    out_f.at[c, pl.ds(off, _RPT)])
    pltpu.sync_copy(acc.at[pl.ds(off, _RPT), pl.ds(_D, _DG)],
                    out_d.at[c, pl.ds(off, _RPT)])


_RB = 1000


def _tc_body(x_ref, pf_ref, pd_ref, ws_ref, wn_ref, bias_ref, o_ref):
    p = pf_ref[0] + pf_ref[1]
    deg = jnp.sum(pd_ref[0] + pd_ref[1], axis=1, keepdims=True)
    h = p * (1.0 / jnp.maximum(deg, 1.0))
    o_ref[...] = (
        jnp.dot(x_ref[...], ws_ref[...], preferred_element_type=jnp.float32)
        + jnp.dot(h, wn_ref[...], preferred_element_type=jnp.float32)
        + bias_ref[...]
    )


_tc_dense = pl.pallas_call(
    _tc_body,
    grid=(_N // _RB,),
    in_specs=[
        pl.BlockSpec((_RB, _D), lambda i: (i, 0)),
        pl.BlockSpec((_NC, _RB, _D), lambda i: (0, i, 0)),
        pl.BlockSpec((_NC, _RB, _DG), lambda i: (0, i, 0)),
        pl.BlockSpec((_D, _D), lambda i: (0, 0)),
        pl.BlockSpec((_D, _D), lambda i: (0, 0)),
        pl.BlockSpec((1, _D), lambda i: (0, 0)),
    ],
    out_specs=pl.BlockSpec((_RB, _D), lambda i: (i, 0)),
    out_shape=jax.ShapeDtypeStruct((_N, _D), jnp.float32),
)


def kernel(x, edge_index, W_self, b_self, W_neigh, b_neigh):
    ei = edge_index.astype(jnp.int32)
    pad_src = jnp.zeros((_EP - _E,), jnp.int32)
    pad_dst = _N + jnp.arange(_EP - _E, dtype=jnp.int32) % (_NP - _N)
    src = jnp.concatenate([ei[0], pad_src]).reshape(_NW, _NPAIR, 2, _B)
    dst = jnp.concatenate([ei[1], pad_dst]).reshape(_NW, _NPAIR, 2, _B)
    eidx = jnp.concatenate([src, dst], axis=2)
    xa = jnp.concatenate(
        [x, jnp.ones((_N, 1), jnp.float32), jnp.zeros((_N, _DP - _D - 1), jnp.float32)],
        axis=1,
    )
    pf, pd = _sc_aggregate(xa, eidx)
    bias = (b_self + b_neigh)[None, :]
    return _tc_dense(x, pf, pd, W_self.T, W_neigh.T, bias)


# speedup vs baseline: 2.5683x; 2.5683x over previous
import functools

import jax
import jax.numpy as jnp
from jax import lax
from jax.experimental import pallas as pl
from jax.experimental.pallas import tpu as pltpu
from jax.experimental.pallas import tpu_sc as plsc

_N = 10000
_E = 320000
_D = 128
_DP = 144
_DG = 16
_NC = 2
_NS = 16
_NW = _NC * _NS
_EP = 327680
_EPW = _EP // _NW
_B = 128
_NCH = _EPW // _B
_NPAIR = _NCH // 2
_NP = 10112
_RPT = _NP // _NS

_mesh = plsc.VectorSubcoreMesh(core_axis_name="c", subcore_axis_name="s")


@functools.partial(
    pl.kernel,
    out_type=(
        jax.ShapeDtypeStruct((_NC, _NP, _D), jnp.float32),
        jax.ShapeDtypeStruct((_NC, _NP, _DG), jnp.float32),
    ),
    mesh=_mesh,
    scratch_types=[
        pltpu.VMEM_SHARED((_NP, _DP), jnp.float32),
        pltpu.VMEM((8, _B), jnp.int32),
        pltpu.VMEM((_B, _DP), jnp.float32),
        pltpu.VMEM((_B, _DP), jnp.float32),
        pltpu.SemaphoreType.DMA,
        pltpu.SemaphoreType.DMA,
        pltpu.SemaphoreType.DMA,
        pltpu.SemaphoreType.DMA,
        pltpu.SemaphoreType.DMA,
    ],
    compiler_params=pltpu.CompilerParams(use_tc_tiling_on_sc=False),
)
def _sc_aggregate(xa, eidx, out_f, out_d, acc, idxb, rows0, rows1,
                  gsem0, gsem1, ssem0, ssem1, isem):
    c = lax.axis_index("c")
    s = lax.axis_index("s")
    wid = s * _NC + c
    off = pl.multiple_of(s * _RPT, 8)

    zero16 = jnp.zeros((16,), jnp.float32)

    def zrows_row(r, carry):
        for k in range(_DP // 16):
            rows0[r, pl.ds(16 * k, 16)] = zero16
        return carry

    lax.fori_loop(0, _B, zrows_row, 0)

    def zero_blk(z, carry):
        zoff = pl.multiple_of(off + z * _B, 8)
        pltpu.sync_copy(rows0, acc.at[pl.ds(zoff, _B)])
        return carry

    lax.fori_loop(0, 4, zero_blk, 0)
    tail = pl.multiple_of(off + 4 * _B, 8)
    pltpu.sync_copy(rows0.at[pl.ds(0, _RPT - 4 * _B)],
                    acc.at[pl.ds(tail, _RPT - 4 * _B)])
    plsc.subcore_barrier()

    pltpu.async_copy(eidx.at[wid, 0], idxb.at[pl.ds(0, 4)], isem).wait()
    pltpu.async_copy(xa.at[idxb.at[0]], rows0, gsem0)
    pltpu.async_copy(xa.at[idxb.at[1]], rows1, gsem1)
    pltpu.async_copy(eidx.at[wid, 1], idxb.at[pl.ds(4, 4)], isem)

    def body(i, carry):
        p = lax.rem(i, 2)
        b0 = 4 * p
        bn = 4 * (1 - p)
        pltpu.make_async_copy(xa.at[idxb.at[b0]], rows0, gsem0).wait()
        pltpu.async_copy(rows0, acc.at[idxb.at[b0 + 2]], ssem0, add=True)
        pltpu.make_async_copy(xa.at[idxb.at[b0 + 1]], rows1, gsem1).wait()
        pltpu.async_copy(rows1, acc.at[idxb.at[b0 + 3]], ssem1, add=True)

        pltpu.make_async_copy(rows0, acc.at[idxb.at[b0 + 2]], ssem0).wait()

        @pl.when(i < _NPAIR - 1)
        def _():
            pltpu.make_async_copy(
                eidx.at[wid, i + 1], idxb.at[pl.ds(bn, 4)], isem).wait()
            pltpu.async_copy(xa.at[idxb.at[bn]], rows0, gsem0)

        pltpu.make_async_copy(rows1, acc.at[idxb.at[b0 + 3]], ssem1).wait()

        @pl.when(i < _NPAIR - 1)
        def _():
            pltpu.async_copy(xa.at[idxb.at[bn + 1]], rows1, gsem1)

        @pl.when(i < _NPAIR - 2)
        def _():
            pltpu.async_copy(eidx.at[wid, i + 2], idxb.at[pl.ds(b0, 4)], isem)

        return carry

    lax.fori_loop(0, _NPAIR, body, 0)
    plsc.subcore_barrier()
    pltpu.sync_copy(acc.at[pl.ds(off, _RPT), pl.ds(0, _D)],
                    out_f.at[c, pl.ds(off, _RPT)])
    pltpu.sync_copy(acc.at[pl.ds(off, _RPT), pl.ds(_D, _DG)],
                    out_d.at[c, pl.ds(off, _RPT)])


_RB = 1000


def _tc_body(x_ref, pf_ref, pd_ref, ws_ref, wn_ref, bias_ref, o_ref):
    p = pf_ref[0] + pf_ref[1]
    deg = jnp.sum(pd_ref[0] + pd_ref[1], axis=1, keepdims=True)
    h = p * (1.0 / jnp.maximum(deg, 1.0))
    o_ref[...] = (
        jnp.dot(x_ref[...], ws_ref[...], preferred_element_type=jnp.float32)
        + jnp.dot(h, wn_ref[...], preferred_element_type=jnp.float32)
        + bias_ref[...]
    )


_tc_dense = pl.pallas_call(
    _tc_body,
    grid=(_N // _RB,),
    in_specs=[
        pl.BlockSpec((_RB, _D), lambda i: (i, 0)),
        pl.BlockSpec((_NC, _RB, _D), lambda i: (0, i, 0)),
        pl.BlockSpec((_NC, _RB, _DG), lambda i: (0, i, 0)),
        pl.BlockSpec((_D, _D), lambda i: (0, 0)),
        pl.BlockSpec((_D, _D), lambda i: (0, 0)),
        pl.BlockSpec((1, _D), lambda i: (0, 0)),
    ],
    out_specs=pl.BlockSpec((_RB, _D), lambda i: (i, 0)),
    out_shape=jax.ShapeDtypeStruct((_N, _D), jnp.float32),
)


def kernel(x, edge_index, W_self, b_self, W_neigh, b_neigh):
    ei = edge_index.astype(jnp.int32)
    pad_src = _N + jnp.arange(_EP - _E, dtype=jnp.int32) % (_NP - _N)
    pad_dst = (jnp.arange(_EP - _E, dtype=jnp.int32) * 89) % _N
    src = jnp.concatenate([ei[0], pad_src]).reshape(_NW, _NPAIR, 2, _B)
    dst = jnp.concatenate([ei[1], pad_dst]).reshape(_NW, _NPAIR, 2, _B)
    eidx = jnp.concatenate([src, dst], axis=2)
    xa = jnp.concatenate(
        [x, jnp.ones((_N, 1), jnp.float32), jnp.zeros((_N, _DP - _D - 1), jnp.float32)],
        axis=1,
    )
    xa = jnp.concatenate([xa, jnp.zeros((_NP - _N, _DP), jnp.float32)], axis=0)
    pf, pd = _sc_aggregate(xa, eidx)
    bias = (b_self + b_neigh)[None, :]
    return _tc_dense(x, pf, pd, W_self.T, W_neigh.T, bias)


# speedup vs baseline: 3.1160x; 1.2133x over previous
import functools

import jax
import jax.numpy as jnp
from jax import lax
from jax.experimental import pallas as pl
from jax.experimental.pallas import tpu as pltpu
from jax.experimental.pallas import tpu_sc as plsc

_N = 10000
_E = 320000
_D = 128
_DG = 16
_NC = 2
_NS = 16
_NW = _NC * _NS
_EP = 327680
_EPW = _EP // _NW
_B = 128
_NCH = _EPW // _B
_NPAIR = _NCH // 2
_NP = 10112
_RPT = _NP // _NS

_mesh = plsc.VectorSubcoreMesh(core_axis_name="c", subcore_axis_name="s")


@functools.partial(
    pl.kernel,
    out_type=(
        jax.ShapeDtypeStruct((_NC, _NP, _D), jnp.float32),
        jax.ShapeDtypeStruct((_NC, _NP, _DG), jnp.float32),
    ),
    mesh=_mesh,
    scratch_types=[
        pltpu.VMEM_SHARED((_NP, _D), jnp.float32),
        pltpu.VMEM_SHARED((_NP, _DG), jnp.float32),
        pltpu.VMEM((12, _B), jnp.int32),
        pltpu.VMEM((_B, _D), jnp.float32),
        pltpu.VMEM((_B, _D), jnp.float32),
        pltpu.VMEM((_B, _DG), jnp.float32),
        pltpu.SemaphoreType.DMA,
        pltpu.SemaphoreType.DMA,
        pltpu.SemaphoreType.DMA,
        pltpu.SemaphoreType.DMA,
        pltpu.SemaphoreType.DMA,
        pltpu.SemaphoreType.DMA,
    ],
    compiler_params=pltpu.CompilerParams(use_tc_tiling_on_sc=False),
)
def _sc_aggregate(xp, eidx, out_f, out_d, acc, dacc, idxb, rows0, rows1, ones_v,
                  gsem0, gsem1, ssem0, ssem1, dsem, isem):
    c = lax.axis_index("c")
    s = lax.axis_index("s")
    wid = s * _NC + c
    off = pl.multiple_of(s * _RPT, 8)

    zero16 = jnp.zeros((16,), jnp.float32)

    def zrows_row(r, carry):
        for k in range(_D // 16):
            rows0[r, pl.ds(16 * k, 16)] = zero16
        ones_v[r, pl.ds(0, 16)] = jnp.full((16,), 1.0 / _DG, jnp.float32)
        return carry

    lax.fori_loop(0, _B, zrows_row, 0)

    def zero_blk(z, carry):
        zoff = pl.multiple_of(off + z * _B, 8)
        pltpu.sync_copy(rows0, acc.at[pl.ds(zoff, _B)])
        pltpu.sync_copy(rows0.at[pl.ds(0, _B), pl.ds(0, _DG)],
                        dacc.at[pl.ds(zoff, _B)])
        return carry

    lax.fori_loop(0, 4, zero_blk, 0)
    tail = pl.multiple_of(off + 4 * _B, 8)
    pltpu.sync_copy(rows0.at[pl.ds(0, _RPT - 4 * _B)],
                    acc.at[pl.ds(tail, _RPT - 4 * _B)])
    pltpu.sync_copy(rows0.at[pl.ds(0, _RPT - 4 * _B), pl.ds(0, _DG)],
                    dacc.at[pl.ds(tail, _RPT - 4 * _B)])
    plsc.subcore_barrier()

    pltpu.async_copy(eidx.at[wid, 0], idxb.at[pl.ds(0, 6)], isem).wait()
    pltpu.async_copy(xp.at[idxb.at[0]], rows0, gsem0)
    pltpu.async_copy(xp.at[idxb.at[1]], rows1, gsem1)
    pltpu.async_copy(eidx.at[wid, 1], idxb.at[pl.ds(6, 6)], isem)

    def body(i, carry):
        p = lax.rem(i, 2)
        b0 = 6 * p
        bn = 6 * (1 - p)
        pltpu.make_async_copy(xp.at[idxb.at[b0]], rows0, gsem0).wait()
        pltpu.async_copy(rows0, acc.at[idxb.at[b0 + 2]], ssem0, add=True)
        pltpu.async_copy(ones_v, dacc.at[idxb.at[b0 + 4]], dsem, add=True)
        pltpu.make_async_copy(xp.at[idxb.at[b0 + 1]], rows1, gsem1).wait()
        pltpu.async_copy(rows1, acc.at[idxb.at[b0 + 3]], ssem1, add=True)
        pltpu.async_copy(ones_v, dacc.at[idxb.at[b0 + 5]], dsem, add=True)

        pltpu.make_async_copy(rows0, acc.at[idxb.at[b0 + 2]], ssem0).wait()

        @pl.when(i < _NPAIR - 1)
        def _():
            pltpu.make_async_copy(
                eidx.at[wid, i + 1], idxb.at[pl.ds(bn, 6)], isem).wait()
            pltpu.async_copy(xp.at[idxb.at[bn]], rows0, gsem0)

        pltpu.make_async_copy(rows1, acc.at[idxb.at[b0 + 3]], ssem1).wait()

        @pl.when(i < _NPAIR - 1)
        def _():
            pltpu.async_copy(xp.at[idxb.at[bn + 1]], rows1, gsem1)

        pltpu.make_async_copy(ones_v, dacc.at[idxb.at[b0 + 4]], dsem).wait()
        pltpu.make_async_copy(ones_v, dacc.at[idxb.at[b0 + 5]], dsem).wait()

        @pl.when(i < _NPAIR - 2)
        def _():
            pltpu.async_copy(eidx.at[wid, i + 2], idxb.at[pl.ds(b0, 6)], isem)

        return carry

    lax.fori_loop(0, _NPAIR, body, 0)
    plsc.subcore_barrier()
    pltpu.sync_copy(acc.at[pl.ds(off, _RPT)], out_f.at[c, pl.ds(off, _RPT)])
    pltpu.sync_copy(dacc.at[pl.ds(off, _RPT)], out_d.at[c, pl.ds(off, _RPT)])


_RB = 1000


def _tc_body(x_ref, pf_ref, pd_ref, ws_ref, wn_ref, bias_ref, o_ref):
    p = pf_ref[0] + pf_ref[1]
    deg = jnp.sum(pd_ref[0] + pd_ref[1], axis=1, keepdims=True)
    h = p * (1.0 / jnp.maximum(deg, 1.0))
    o_ref[...] = (
        jnp.dot(x_ref[...], ws_ref[...], preferred_element_type=jnp.float32)
        + jnp.dot(h, wn_ref[...], preferred_element_type=jnp.float32)
        + bias_ref[...]
    )


_tc_dense = pl.pallas_call(
    _tc_body,
    grid=(_N // _RB,),
    in_specs=[
        pl.BlockSpec((_RB, _D), lambda i: (i, 0)),
        pl.BlockSpec((_NC, _RB, _D), lambda i: (0, i, 0)),
        pl.BlockSpec((_NC, _RB, _DG), lambda i: (0, i, 0)),
        pl.BlockSpec((_D, _D), lambda i: (0, 0)),
        pl.BlockSpec((_D, _D), lambda i: (0, 0)),
        pl.BlockSpec((1, _D), lambda i: (0, 0)),
    ],
    out_specs=pl.BlockSpec((_RB, _D), lambda i: (i, 0)),
    out_shape=jax.ShapeDtypeStruct((_N, _D), jnp.float32),
)


def kernel(x, edge_index, W_self, b_self, W_neigh, b_neigh):
    ei = edge_index.astype(jnp.int32)
    npad = _EP - _E
    j = jnp.arange(npad, dtype=jnp.int32)
    pad_src = _N + j % (_NP - _N)
    pad_dst = (j * 89) % _N
    pad_deg = _N + j % (_NP - _N)
    src = jnp.concatenate([ei[0], pad_src]).reshape(_NW, _NPAIR, 2, _B)
    dst = jnp.concatenate([ei[1], pad_dst]).reshape(_NW, _NPAIR, 2, _B)
    dd = jnp.concatenate([ei[1], pad_deg]).reshape(_NW, _NPAIR, 2, _B)
    eidx = jnp.concatenate([src, dst, dd], axis=2)
    xp = jnp.concatenate([x, jnp.zeros((_NP - _N, _D), jnp.float32)], axis=0)
    pf, pd = _sc_aggregate(xp, eidx)
    bias = (b_self + b_neigh)[None, :]
    return _tc_dense(x, pf, pd, W_self.T, W_neigh.T, bias)
